# Initial kernel scaffold; baseline (speedup 1.0000x reference)
import jax
import jax.numpy as jnp
from jax.experimental import pallas as pl


def kernel(x, w_proj0, b_proj0, w_proj1, b_proj1, pos0, pos1, ln1_g, ln1_b, wq, bq, wk, bk, wv, bv, wo, bo, ln2_g, ln2_b, w_ff1, b_ff1, w_ff2, b_ff2, ln_g, ln_b, w_fus1, b_fus1, w_fus2, b_fus2):
    raise NotImplementedError("write your pallas kernel here")


# speedup vs baseline: 3.5700x; 3.5700x over previous
import numpy as np
import jax
import jax.numpy as jnp
from jax.experimental import pallas as pl
from jax.experimental.pallas import tpu as pltpu

_B, _L, _C = 8, 128, 8
_H, _NH, _NL = 256, 8, 2
_DH = _H // _NH
_FF = 4 * _H
_PLS = (8, 16)
_STS = (4, 8)
_MAXPAD = _PLS[1] - 1
_LP = _L + _MAXPAD
_R = _B * _L
_ISCALE = _DH ** -0.5


def _dot(a, b):
    return jnp.dot(a, b, preferred_element_type=jnp.float32)


def _dott(a, b):
    return jax.lax.dot_general(a, b, (((1,), (1,)), ((), ())),
                               preferred_element_type=jnp.float32)


def _ln_rows(x, g, b):
    mu = jnp.mean(x, -1, keepdims=True)
    xc = x - mu
    var = jnp.mean(xc * xc, -1, keepdims=True)
    return xc * jax.lax.rsqrt(var + 1e-5) * g + b


def _gelu(x):
    return 0.5 * x * (1.0 + jax.lax.erf(x * 0.7071067811865476))


def _scale_meta(si):
    pl_len, st = _PLS[si], _STS[si]
    Nmax = (_L - 1 + _MAXPAD - (pl_len - 1)) // st + 1
    Np = ((Nmax + 7) // 8) * 8
    Mg = _L // st
    moff = (_MAXPAD - (pl_len - 1)) // st
    mpref = np.arange(Mg) + moff
    j = np.arange(Np)
    T1 = (j[None, :] <= mpref[:, None]).astype(np.float32)
    addm2 = np.where(j[None, :] <= mpref[:, None], 0.0, -1e9).astype(np.float32)
    dsel = (j[None, :] == mpref[:, None]).astype(np.float32)
    nck = Mg // 8
    kexts = [min(Np, ((8 * c + 7 + moff) // 8 + 1) * 8) for c in range(nck)]
    return pl_len, st, Nmax, Np, Mg, moff, T1, addm2, dsel, kexts


def _head_group_matrix():
    g = np.zeros((_H, _NH), np.float32)
    for h in range(_NH):
        g[h * _DH:(h + 1) * _DH, h] = 1.0
    return g


def _make_scale_body(si):
    pl_len, st, Nmax, Np, Mg, moff, T1_np, addm2_np, dsel_np, kexts = _scale_meta(si)
    plC = pl_len * _C

    def body(win_ref, pos_ref, t1_ref, addm2_ref, dsel_ref, gh_ref,
             wp_ref, bp_ref,
             ln1g_ref, ln1b_ref, wq_ref, bq_ref, wk_ref, bk_ref,
             wv_ref, bv_ref, wo_ref, bo_ref, ln2g_ref, ln2b_ref,
             wf1_ref, bf1_ref, wf2_ref, bf2_ref, out_ref):
        T1 = t1_ref[...]
        addm2 = addm2_ref[...]
        dsel = dsel_ref[...]
        gh = gh_ref[...]
        ln1g = ln1g_ref[...]
        ln1b = ln1b_ref[...]
        ln2g = ln2g_ref[...]
        ln2b = ln2b_ref[...]
        bq = bq_ref[...]
        bk = bk_ref[...]
        bv = bv_ref[...]
        bo = bo_ref[...]
        bf1 = bf1_ref[...]
        bf2 = bf2_ref[...]

        win = win_ref[...].reshape(Np, plC)
        seq1 = _dot(win, wp_ref[...]) + bp_ref[...] + pos_ref[...]

        s2 = _ln_rows(seq1, ln1g[0:1, :], ln1b[0:1, :])
        q = _dot(s2, wq_ref[0]) + bq[0:1, :]
        k = _dot(s2, wk_ref[0]) + bk[0:1, :]
        v = _dot(s2, wv_ref[0]) + bv[0:1, :]
        Es = []
        for h in range(_NH):
            sl = slice(h * _DH, (h + 1) * _DH)
            Es.append(jnp.exp(_dott(q[:, sl], k[:, sl]) * _ISCALE))

        for c, ke in enumerate(kexts):
            ms = slice(8 * c, 8 * c + 8)
            T1c = T1[ms]
            rows = 8 * ke
            o1parts = []
            for h in range(_NH):
                sl = slice(h * _DH, (h + 1) * _DH)
                F = (Es[h][None, :ke, :] * T1c[:, None, :]).reshape(rows, Np)
                d = jnp.sum(F, axis=-1, keepdims=True)
                o1parts.append(_dot(F, v[:, sl]) / d)
            o1 = jnp.concatenate(o1parts, axis=-1)
            seq_c = jnp.broadcast_to(seq1[None, :ke, :], (8, ke, _H)).reshape(rows, _H)
            seq_c = seq_c + _dot(o1, wo_ref[0]) + bo[0:1, :]
            t2 = _ln_rows(seq_c, ln2g[0:1, :], ln2b[0:1, :])
            h1 = _gelu(_dot(t2, wf1_ref[0]) + bf1[0:1, :])
            seq_c = seq_c + _dot(h1, wf2_ref[0]) + bf2[0:1, :]

            s2b = _ln_rows(seq_c, ln1g[1:2, :], ln1b[1:2, :])
            k2 = _dot(s2b, wk_ref[1]) + bk[1:2, :]
            v2 = _dot(s2b, wv_ref[1]) + bv[1:2, :]
            dsel_c = dsel[ms, :ke]
            s2b3 = s2b.reshape(8, ke, _H)
            qpre = jnp.sum(s2b3 * dsel_c[:, :, None], axis=1)
            q2 = _dot(qpre, wq_ref[1]) + bq[1:2, :]
            prod = (k2.reshape(8, ke, _H) * q2[:, None, :]).reshape(rows, _H)
            sc2 = _dot(prod, gh) * _ISCALE
            sc23 = sc2.reshape(8, ke, _NH) + addm2[ms, :ke][:, :, None]
            m2 = jnp.max(sc23, axis=1, keepdims=True)
            e2 = jnp.exp(sc23 - m2)
            p2 = e2 / jnp.sum(e2, axis=1, keepdims=True)
            v23 = v2.reshape(8, ke, _H)
            o2parts = []
            for h in range(_NH):
                sl = slice(h * _DH, (h + 1) * _DH)
                o2parts.append(jnp.sum(v23[:, :, sl] * p2[:, :, h:h + 1], axis=1))
            o2 = jnp.concatenate(o2parts, axis=-1)
            seq_last = jnp.sum(seq_c.reshape(8, ke, _H) * dsel_c[:, :, None], axis=1)
            seq_last = seq_last + _dot(o2, wo_ref[1]) + bo[1:2, :]
            t2b = _ln_rows(seq_last, ln2g[1:2, :], ln2b[1:2, :])
            h2 = _gelu(_dot(t2b, wf1_ref[1]) + bf1[1:2, :])
            seq_last = seq_last + _dot(h2, wf2_ref[1]) + bf2[1:2, :]
            out_ref[0, ms, :] = seq_last

    return body


def _full_spec(shape):
    nd = len(shape)
    return pl.BlockSpec(shape, lambda i: (0,) * nd)


def _run_scale(si, x_padded, w_proj, b_proj, pos, ln1_g, ln1_b, wq, bq, wk, bk,
               wv, bv, wo, bo, ln2_g, ln2_b, w_ff1, b_ff1, w_ff2, b_ff2):
    pl_len, st, Nmax, Np, Mg, moff, T1, addm2, dsel, kexts = _scale_meta(si)
    plC = pl_len * _C
    NG = _B * st
    gidx = (np.arange(st)[:, None, None] + st * np.arange(Np)[None, :, None]
            + np.arange(pl_len)[None, None, :])
    gidx = np.minimum(gidx, _LP - 1)
    win = x_padded[:, gidx, :].reshape(NG, Np, plC)
    posp = jnp.zeros((Np, _H), jnp.float32).at[:Nmax].set(pos[:Nmax])
    consts = (jnp.asarray(T1), jnp.asarray(addm2), jnp.asarray(dsel),
              jnp.asarray(_head_group_matrix()))
    weights = (w_proj, b_proj, ln1_g, ln1_b, wq, bq, wk, bk, wv, bv,
               wo, bo, ln2_g, ln2_b, w_ff1, b_ff1, w_ff2, b_ff2)
    in_specs = [
        pl.BlockSpec((1, Np, plC), lambda i: (i, 0, 0)),
        pl.BlockSpec((Np, _H), lambda i: (0, 0)),
    ] + [_full_spec(c.shape) for c in consts] + [_full_spec(w.shape) for w in weights]
    out = pl.pallas_call(
        _make_scale_body(si),
        grid=(NG,),
        in_specs=in_specs,
        out_specs=pl.BlockSpec((1, Mg, _H), lambda i: (i, 0, 0)),
        out_shape=jax.ShapeDtypeStruct((NG, Mg, _H), jnp.float32),
        compiler_params=pltpu.CompilerParams(
            dimension_semantics=("parallel",),
            vmem_limit_bytes=100 * 1024 * 1024,
        ),
    )(win, posp, *consts, *weights)
    return out.reshape(_B, st, Mg, _H).transpose(0, 2, 1, 3).reshape(_R, _H)


def _fusion_body(e0_ref, e1_ref, w1a_ref, w1b_ref, b1_ref, w2_ref, b2_ref,
                 lg_ref, lb_ref, out_ref):
    hpre = (_dot(e0_ref[...], w1a_ref[...])
            + _dot(e1_ref[...], w1b_ref[...])
            + b1_ref[...])
    f = _dot(_gelu(hpre), w2_ref[...]) + b2_ref[...]
    out_ref[...] = _ln_rows(f, lg_ref[...], lb_ref[...])


def kernel(x, w_proj0, b_proj0, w_proj1, b_proj1, pos0, pos1, ln1_g, ln1_b,
           wq, bq, wk, bk, wv, bv, wo, bo, ln2_g, ln2_b, w_ff1, b_ff1,
           w_ff2, b_ff2, ln_g, ln_b, w_fus1, b_fus1, w_fus2, b_fus2):
    x_padded = jnp.concatenate([jnp.zeros((_B, _MAXPAD, _C), x.dtype), x], axis=1)
    common = (ln1_g, ln1_b, wq, bq, wk, bk, wv, bv, wo, bo, ln2_g, ln2_b,
              w_ff1, b_ff1, w_ff2, b_ff2)
    e0 = _run_scale(0, x_padded, w_proj0, b_proj0.reshape(1, _H), pos0, *common)
    e1 = _run_scale(1, x_padded, w_proj1, b_proj1.reshape(1, _H), pos1, *common)
    SBf = 256
    fw = (w_fus1[:_H], w_fus1[_H:],
          b_fus1.reshape(1, _H), w_fus2,
          b_fus2.reshape(1, _H), ln_g.reshape(1, _H), ln_b.reshape(1, _H))
    out = pl.pallas_call(
        _fusion_body,
        grid=(_R // SBf,),
        in_specs=[pl.BlockSpec((SBf, _H), lambda i: (i, 0)),
                  pl.BlockSpec((SBf, _H), lambda i: (i, 0))]
                 + [_full_spec(w.shape) for w in fw],
        out_specs=pl.BlockSpec((SBf, _H), lambda i: (i, 0)),
        out_shape=jax.ShapeDtypeStruct((_R, _H), jnp.float32),
        compiler_params=pltpu.CompilerParams(
            dimension_semantics=("parallel",),
            vmem_limit_bytes=64 * 1024 * 1024,
        ),
    )(e0, e1, *fw)
    return out.reshape(_B, _L, _H)


# speedup vs baseline: 3.7117x; 1.0397x over previous
import numpy as np
import jax
import jax.numpy as jnp
from jax.experimental import pallas as pl
from jax.experimental.pallas import tpu as pltpu

_B, _L, _C = 8, 128, 8
_H, _NH, _NL = 256, 8, 2
_DH = _H // _NH
_FF = 4 * _H
_PLS = (8, 16)
_STS = (4, 8)
_MAXPAD = _PLS[1] - 1
_LP = _L + _MAXPAD
_R = _B * _L
_ISCALE = _DH ** -0.5


def _dot(a, b):
    return jnp.dot(a, b, preferred_element_type=jnp.float32)


def _dott(a, b):
    return jax.lax.dot_general(a, b, (((1,), (1,)), ((), ())),
                               preferred_element_type=jnp.float32)


def _ln_rows(x, g, b):
    mu = jnp.mean(x, -1, keepdims=True)
    xc = x - mu
    var = jnp.mean(xc * xc, -1, keepdims=True)
    return xc * jax.lax.rsqrt(var + 1e-5) * g + b


def _gelu(x):
    return 0.5 * x * (1.0 + jax.lax.erf(x * 0.7071067811865476))


def _scale_meta(si):
    pl_len, st = _PLS[si], _STS[si]
    Nmax = (_L - 1 + _MAXPAD - (pl_len - 1)) // st + 1
    Np = ((Nmax + 7) // 8) * 8
    Mg = _L // st
    moff = (_MAXPAD - (pl_len - 1)) // st
    mpref = np.arange(Mg) + moff
    j = np.arange(Np)
    T1 = (j[None, :] <= mpref[:, None]).astype(np.float32)
    addm2 = np.where(j[None, :] <= mpref[:, None], 0.0, -1e9).astype(np.float32)
    dsel = (j[None, :] == mpref[:, None]).astype(np.float32)
    nck = Mg // 8
    kexts = [min(Np, ((8 * c + 7 + moff) // 8 + 1) * 8) for c in range(nck)]
    return pl_len, st, Nmax, Np, Mg, moff, T1, addm2, dsel, kexts


def _head_group_matrix():
    g = np.zeros((_H, _NH), np.float32)
    for h in range(_NH):
        g[h * _DH:(h + 1) * _DH, h] = 1.0
    return g


def _make_scale_body(si, gpb):
    pl_len, st, Nmax, Np, Mg, moff, T1_np, addm2_np, dsel_np, kexts = _scale_meta(si)
    plC = pl_len * _C

    def body(win_ref, pos_ref, t1_ref, addm2_ref, dsel_ref, gh_ref,
             wp_ref, bp_ref,
             ln1g_ref, ln1b_ref, wq_ref, bq_ref, wk_ref, bk_ref,
             wv_ref, bv_ref, wo_ref, bo_ref, ln2g_ref, ln2b_ref,
             wf1_ref, bf1_ref, wf2_ref, bf2_ref, out_ref):
        T1 = t1_ref[...]
        addm2 = addm2_ref[...]
        dsel = dsel_ref[...]
        gh = gh_ref[...]
        ln1g = ln1g_ref[...]
        ln1b = ln1b_ref[...]
        ln2g = ln2g_ref[...]
        ln2b = ln2b_ref[...]
        bq = bq_ref[...]
        bk = bk_ref[...]
        bv = bv_ref[...]
        bo = bo_ref[...]
        bf1 = bf1_ref[...]
        bf2 = bf2_ref[...]
        pos = pos_ref[...]

        for g in range(gpb):
            win = win_ref[g]
            seq1 = _dot(win, wp_ref[...]) + bp_ref[...] + pos

            s2 = _ln_rows(seq1, ln1g[0:1, :], ln1b[0:1, :])
            q = _dot(s2, wq_ref[0]) + bq[0:1, :]
            k = _dot(s2, wk_ref[0]) + bk[0:1, :]
            v = _dot(s2, wv_ref[0]) + bv[0:1, :]
            Es = []
            for h in range(_NH):
                sl = slice(h * _DH, (h + 1) * _DH)
                Es.append(jnp.exp(_dott(q[:, sl], k[:, sl]) * _ISCALE))

            for c, ke in enumerate(kexts):
                ms = slice(8 * c, 8 * c + 8)
                T1c = T1[ms]
                rows = 8 * ke
                o1parts = []
                for h in range(_NH):
                    sl = slice(h * _DH, (h + 1) * _DH)
                    F = (Es[h][None, :ke, :] * T1c[:, None, :]).reshape(rows, Np)
                    d = jnp.sum(F, axis=-1, keepdims=True)
                    o1parts.append(_dot(F, v[:, sl]) / d)
                o1 = jnp.concatenate(o1parts, axis=-1)
                seq_c = jnp.broadcast_to(seq1[None, :ke, :], (8, ke, _H)).reshape(rows, _H)
                seq_c = seq_c + _dot(o1, wo_ref[0]) + bo[0:1, :]
                t2 = _ln_rows(seq_c, ln2g[0:1, :], ln2b[0:1, :])
                h1 = _gelu(_dot(t2, wf1_ref[0]) + bf1[0:1, :])
                seq_c = seq_c + _dot(h1, wf2_ref[0]) + bf2[0:1, :]

                s2b = _ln_rows(seq_c, ln1g[1:2, :], ln1b[1:2, :])
                k2 = _dot(s2b, wk_ref[1]) + bk[1:2, :]
                v2 = _dot(s2b, wv_ref[1]) + bv[1:2, :]
                dsel_c = dsel[ms, :ke]
                s2b3 = s2b.reshape(8, ke, _H)
                qpre = jnp.sum(s2b3 * dsel_c[:, :, None], axis=1)
                q2 = _dot(qpre, wq_ref[1]) + bq[1:2, :]
                prod = (k2.reshape(8, ke, _H) * q2[:, None, :]).reshape(rows, _H)
                sc2 = _dot(prod, gh) * _ISCALE
                sc23 = sc2.reshape(8, ke, _NH) + addm2[ms, :ke][:, :, None]
                m2 = jnp.max(sc23, axis=1, keepdims=True)
                e2 = jnp.exp(sc23 - m2)
                p2 = e2 / jnp.sum(e2, axis=1, keepdims=True)
                v23 = v2.reshape(8, ke, _H)
                o2parts = []
                for h in range(_NH):
                    sl = slice(h * _DH, (h + 1) * _DH)
                    o2parts.append(jnp.sum(v23[:, :, sl] * p2[:, :, h:h + 1], axis=1))
                o2 = jnp.concatenate(o2parts, axis=-1)
                seq_last = jnp.sum(seq_c.reshape(8, ke, _H) * dsel_c[:, :, None], axis=1)
                seq_last = seq_last + _dot(o2, wo_ref[1]) + bo[1:2, :]
                t2b = _ln_rows(seq_last, ln2g[1:2, :], ln2b[1:2, :])
                h2 = _gelu(_dot(t2b, wf1_ref[1]) + bf1[1:2, :])
                seq_last = seq_last + _dot(h2, wf2_ref[1]) + bf2[1:2, :]
                out_ref[g, ms, :] = seq_last

    return body


def _full_spec(shape):
    nd = len(shape)
    return pl.BlockSpec(shape, lambda i: (0,) * nd)


def _run_scale(si, gpb, x_padded, w_proj, b_proj, pos, ln1_g, ln1_b, wq, bq,
               wk, bk, wv, bv, wo, bo, ln2_g, ln2_b, w_ff1, b_ff1, w_ff2, b_ff2):
    pl_len, st, Nmax, Np, Mg, moff, T1, addm2, dsel, kexts = _scale_meta(si)
    plC = pl_len * _C
    NG = _B * st
    gidx = (np.arange(st)[:, None, None] + st * np.arange(Np)[None, :, None]
            + np.arange(pl_len)[None, None, :])
    gidx = np.minimum(gidx, _LP - 1)
    win = x_padded[:, gidx, :].reshape(NG, Np, plC)
    posp = jnp.zeros((Np, _H), jnp.float32).at[:Nmax].set(pos[:Nmax])
    consts = (jnp.asarray(T1), jnp.asarray(addm2), jnp.asarray(dsel),
              jnp.asarray(_head_group_matrix()))
    weights = (w_proj, b_proj, ln1_g, ln1_b, wq, bq, wk, bk, wv, bv,
               wo, bo, ln2_g, ln2_b, w_ff1, b_ff1, w_ff2, b_ff2)
    in_specs = [
        pl.BlockSpec((gpb, Np, plC), lambda i: (i, 0, 0)),
        pl.BlockSpec((Np, _H), lambda i: (0, 0)),
    ] + [_full_spec(c.shape) for c in consts] + [_full_spec(w.shape) for w in weights]
    out = pl.pallas_call(
        _make_scale_body(si, gpb),
        grid=(NG // gpb,),
        in_specs=in_specs,
        out_specs=pl.BlockSpec((gpb, Mg, _H), lambda i: (i, 0, 0)),
        out_shape=jax.ShapeDtypeStruct((NG, Mg, _H), jnp.float32),
        compiler_params=pltpu.CompilerParams(
            dimension_semantics=("parallel",),
            vmem_limit_bytes=100 * 1024 * 1024,
        ),
    )(win, posp, *consts, *weights)
    return out.reshape(_B, st, Mg, _H).transpose(0, 2, 1, 3).reshape(_R, _H)


def _fusion_body(e0_ref, e1_ref, w1a_ref, w1b_ref, b1_ref, w2_ref, b2_ref,
                 lg_ref, lb_ref, out_ref):
    hpre = (_dot(e0_ref[...], w1a_ref[...])
            + _dot(e1_ref[...], w1b_ref[...])
            + b1_ref[...])
    f = _dot(_gelu(hpre), w2_ref[...]) + b2_ref[...]
    out_ref[...] = _ln_rows(f, lg_ref[...], lb_ref[...])


def kernel(x, w_proj0, b_proj0, w_proj1, b_proj1, pos0, pos1, ln1_g, ln1_b,
           wq, bq, wk, bk, wv, bv, wo, bo, ln2_g, ln2_b, w_ff1, b_ff1,
           w_ff2, b_ff2, ln_g, ln_b, w_fus1, b_fus1, w_fus2, b_fus2):
    x_padded = jnp.concatenate([jnp.zeros((_B, _MAXPAD, _C), x.dtype), x], axis=1)
    common = (ln1_g, ln1_b, wq, bq, wk, bk, wv, bv, wo, bo, ln2_g, ln2_b,
              w_ff1, b_ff1, w_ff2, b_ff2)
    e0 = _run_scale(0, 2, x_padded, w_proj0, b_proj0.reshape(1, _H), pos0, *common)
    e1 = _run_scale(1, 4, x_padded, w_proj1, b_proj1.reshape(1, _H), pos1, *common)
    SBf = 256
    fw = (w_fus1[:_H], w_fus1[_H:],
          b_fus1.reshape(1, _H), w_fus2,
          b_fus2.reshape(1, _H), ln_g.reshape(1, _H), ln_b.reshape(1, _H))
    out = pl.pallas_call(
        _fusion_body,
        grid=(_R // SBf,),
        in_specs=[pl.BlockSpec((SBf, _H), lambda i: (i, 0)),
                  pl.BlockSpec((SBf, _H), lambda i: (i, 0))]
                 + [_full_spec(w.shape) for w in fw],
        out_specs=pl.BlockSpec((SBf, _H), lambda i: (i, 0)),
        out_shape=jax.ShapeDtypeStruct((_R, _H), jnp.float32),
        compiler_params=pltpu.CompilerParams(
            dimension_semantics=("parallel",),
            vmem_limit_bytes=64 * 1024 * 1024,
        ),
    )(e0, e1, *fw)
    return out.reshape(_B, _L, _H)
